# baseline (speedup 1.0000x reference)
import functools

import jax
import jax.numpy as jnp
from jax import lax
from jax.experimental import pallas as pl
from jax.experimental.pallas import tpu as pltpu

_INTERPRET = False

B, T, H = 1, 2048, 1024
I = 2048
E = 8
K = 2
NT_TOK = B * T
NP = NT_TOK * K
TILE = 256
NT_E = NP // TILE + E
PE = NT_E * TILE
NT_S = NT_TOK // TILE
NT = NT_E + NT_S
PTOT = PE + NT_TOK
EPAD = 128


def _router_kernel(x_ref, wg_ref, e1_ref, e2_ref, v1_ref, v2_ref, aux_ref):
    x = x_ref[...]
    wg = wg_ref[...]
    logits = lax.dot_general(x, wg, (((1,), (1,)), ((), ())),
                             preferred_element_type=jnp.float32)
    col = lax.broadcasted_iota(jnp.int32, logits.shape, 1)
    valid = col < E
    z = jnp.where(valid, logits, -1e30)
    zmax = jnp.max(z, axis=1, keepdims=True)
    p = jnp.where(valid, jnp.exp(z - zmax), 0.0)
    probs = p / jnp.sum(p, axis=1, keepdims=True)
    v1 = jnp.max(probs, axis=1, keepdims=True)
    e1 = jnp.min(jnp.where(probs >= v1, col, EPAD), axis=1, keepdims=True)
    probs2 = jnp.where(col == e1, -1.0, probs)
    v2 = jnp.max(probs2, axis=1, keepdims=True)
    e2 = jnp.min(jnp.where(probs2 >= v2, col, EPAD), axis=1, keepdims=True)
    e1_ref[...] = e1
    e2_ref[...] = e2
    v1_ref[...] = v1
    v2_ref[...] = v2
    cnt = jnp.sum((col == e1).astype(jnp.float32)
                  + (col == e2).astype(jnp.float32), axis=0, keepdims=True)
    imp = jnp.mean(probs, axis=0, keepdims=True)
    aux_ref[...] = jnp.sum(imp * cnt, axis=1, keepdims=True) * (
        float(E) / float(NT_TOK * K))


def _router(x_flat, Wg):
    wg_pad = jnp.zeros((EPAD, H), jnp.float32).at[:E].set(Wg)
    kern = pl.pallas_call(
        _router_kernel,
        out_shape=(
            jax.ShapeDtypeStruct((T, 1), jnp.int32),
            jax.ShapeDtypeStruct((T, 1), jnp.int32),
            jax.ShapeDtypeStruct((T, 1), jnp.float32),
            jax.ShapeDtypeStruct((T, 1), jnp.float32),
            jax.ShapeDtypeStruct((1, 1), jnp.float32),
        ),
        interpret=_INTERPRET,
    )
    return kern(x_flat, wg_pad)


def _mlp_kernel(em_ref, valid_ref, xg_ref, w1_ref, w2_ref, ws_ref, out_ref):
    i = pl.program_id(0)

    @pl.when(valid_ref[i] != 0)
    def _():
        x = xg_ref[...].astype(jnp.bfloat16)
        h = lax.dot_general(x, w1_ref[0], (((1,), (1,)), ((), ())),
                            preferred_element_type=jnp.float32)
        a = h * jax.nn.sigmoid(h)
        o = lax.dot_general(a.astype(jnp.bfloat16), w2_ref[0],
                            (((1,), (1,)), ((), ())),
                            preferred_element_type=jnp.float32)
        out_ref[...] = o * ws_ref[0, 0][:, None]


def _expert_mlp(em, valid, xg, W1all, W2all, wslot):
    grid_spec = pltpu.PrefetchScalarGridSpec(
        num_scalar_prefetch=2,
        grid=(NT,),
        in_specs=[
            pl.BlockSpec((TILE, H), lambda i, em, vd: (i, 0)),
            pl.BlockSpec((1, I, H), lambda i, em, vd: (em[i], 0, 0)),
            pl.BlockSpec((1, H, I), lambda i, em, vd: (em[i], 0, 0)),
            pl.BlockSpec((1, 1, TILE), lambda i, em, vd: (i, 0, 0)),
        ],
        out_specs=pl.BlockSpec((TILE, H), lambda i, em, vd: (i, 0)),
    )
    kern = pl.pallas_call(
        _mlp_kernel,
        grid_spec=grid_spec,
        out_shape=jax.ShapeDtypeStruct((PTOT, H), jnp.float32),
        interpret=_INTERPRET,
    )
    return kern(em, valid, xg, W1all, W2all, wslot.reshape(NT, 1, TILE))


def _route_sort_jax(e1, e2, v1, v2):
    eid = jnp.concatenate([e1, e2])
    wv = jnp.concatenate([v1, v2])
    tok = jnp.concatenate([jnp.arange(NT_TOK, dtype=jnp.int32)] * 2)
    cnt = jnp.bincount(eid, length=E)
    tiles_e = (cnt + TILE - 1) // TILE
    cum_rows = jnp.concatenate([jnp.zeros(1, jnp.int32),
                                jnp.cumsum(tiles_e * TILE).astype(jnp.int32)])
    cum_cnt = jnp.concatenate([jnp.zeros(1, jnp.int32),
                               jnp.cumsum(cnt).astype(jnp.int32)])
    order = jnp.argsort(eid, stable=True)
    eid_sorted = eid[order]
    rank = jnp.arange(NP, dtype=jnp.int32)
    slot_of_rank = cum_rows[eid_sorted] + rank - cum_cnt[eid_sorted]
    pos = jnp.zeros(NP, jnp.int32).at[order].set(slot_of_rank)
    tok_full = jnp.zeros(PTOT, jnp.int32)
    tok_full = tok_full.at[slot_of_rank].set(tok[order])
    tok_full = tok_full.at[PE:].set(jnp.arange(NT_TOK, dtype=jnp.int32))
    wslot = jnp.zeros(PTOT, jnp.float32).at[slot_of_rank].set(wv[order])
    wslot = wslot.at[PE:].set(1.0)
    used_tiles = jnp.sum(tiles_e).astype(jnp.int32)
    cum_tiles = jnp.cumsum(tiles_e).astype(jnp.int32)
    ti = jnp.arange(NT, dtype=jnp.int32)
    em_e = jnp.sum(ti[:, None] >= cum_tiles[None, :], axis=1).astype(jnp.int32)
    em = jnp.where(ti < NT_E, jnp.minimum(em_e, E), E)
    valid = jnp.where(ti < NT_E, (ti < used_tiles).astype(jnp.int32), 1)
    return tok_full, wslot, em, valid, pos[:NT_TOK], pos[NT_TOK:]


def kernel(x, Wg, W1, W2, Ws1, Ws2):
    x_flat = x.reshape(NT_TOK, H)
    e1, e2, v1, v2, aux = _router(x_flat, Wg)
    e1, e2 = e1[:, 0], e2[:, 0]
    v1, v2 = v1[:, 0], v2[:, 0]

    tok_full, wslot, em, valid, pos1, pos2 = _route_sort_jax(e1, e2, v1, v2)

    xg = x_flat[tok_full]

    W1all = jnp.concatenate([W1, Ws1[None]], axis=0).astype(jnp.bfloat16)
    W2all = jnp.concatenate([W2, Ws2[None]], axis=0).astype(jnp.bfloat16)
    o = _expert_mlp(em, valid, xg, W1all, W2all, wslot)

    y_flat = o[pos1] + o[pos2] + o[PE + jnp.arange(NT_TOK)]

    y = y_flat.reshape(B, T, H).astype(x.dtype)
    return y, aux[0, 0]
